# baseline (device time: 131284 ns/iter reference)
import jax
import jax.numpy as jnp
from jax import lax
from jax.experimental import pallas as pl
from jax.experimental.pallas import tpu as pltpu

N_DEV = 4
HQ_PER = 8
DH = 64
BLK = 64


def _allreduce_body(p_ref, out_ref, comm_ref, send_sems, recv_sems):
    my = lax.axis_index("i")
    left = (my - 1) % N_DEV
    right = (my + 1) % N_DEV

    barrier = pltpu.get_barrier_semaphore()
    for nbr in (left, right):
        pl.semaphore_signal(
            barrier, inc=1, device_id=(nbr,), device_id_type=pl.DeviceIdType.MESH
        )
    pl.semaphore_wait(barrier, 2)

    comm_ref[0, :, :] = p_ref[:, :]
    out_ref[:, :] = p_ref[:, :]

    for h in range(N_DEV - 1):
        rdma = pltpu.make_async_remote_copy(
            src_ref=comm_ref.at[h],
            dst_ref=comm_ref.at[h + 1],
            send_sem=send_sems.at[h],
            recv_sem=recv_sems.at[h],
            device_id=(right,),
            device_id_type=pl.DeviceIdType.MESH,
        )
        rdma.start()
        rdma.wait()
        out_ref[:, :] = out_ref[:, :] + comm_ref[h + 1, :, :]


def kernel(x, Wq, K_ext, V_ext, Wo):
    B, Sq, D = x.shape
    my = lax.axis_index("i")

    K = lax.dynamic_slice_in_dim(K_ext, my * HQ_PER, HQ_PER, axis=2)
    V = lax.dynamic_slice_in_dim(V_ext, my * HQ_PER, HQ_PER, axis=2)

    Q = (x @ Wq).reshape(B, Sq, HQ_PER, DH)
    qb = jnp.arange(Sq) // BLK
    mask = qb[None, :] <= qb[:, None]
    scores = jnp.einsum("bihd,bjhd->bhij", Q, K) * 0.125
    scores = jnp.where(mask[None, None], scores, -1e9)
    w = jax.nn.softmax(scores, axis=-1)
    ctx = jnp.einsum("bhij,bjhd->bihd", w, V).reshape(B, Sq, HQ_PER * DH)
    partial = (ctx @ Wo).reshape(B * Sq, D)

    out = pl.pallas_call(
        _allreduce_body,
        out_shape=jax.ShapeDtypeStruct((B * Sq, D), jnp.float32),
        in_specs=[pl.BlockSpec(memory_space=pltpu.VMEM)],
        out_specs=pl.BlockSpec(memory_space=pltpu.VMEM),
        scratch_shapes=[
            pltpu.VMEM((N_DEV, B * Sq, D), jnp.float32),
            pltpu.SemaphoreType.DMA((N_DEV - 1,)),
            pltpu.SemaphoreType.DMA((N_DEV - 1,)),
        ],
        compiler_params=pltpu.CompilerParams(collective_id=0),
    )(partial)
    return out.reshape(B, Sq, D)


# device time: 55961 ns/iter; 2.3460x vs baseline; 2.3460x over previous
import jax
import jax.numpy as jnp
from jax import lax
from jax.experimental import pallas as pl
from jax.experimental.pallas import tpu as pltpu

N_DEV = 4
HQ_PER = 8
DH = 64
BLK = 64
SQ = 512
QROWS = 256


def _body(x_ref, wq_ref, k_ref, v_ref, wo_ref, out_ref,
          ctx_scr, rs1_scr, rs2_scr, s_sems, r_sems):
    my = lax.axis_index("i")
    p1 = my ^ 1
    p2 = 3 - my

    sA = ((my + 1) >> 1) & 1
    qA = (my >> 1) & 1
    sB = (my >> 1) & 1
    qB = my & 1

    k1A = sA * 256
    s1A = (1 - sA) * 256
    k1B = 512 + sB * 256
    s1B = 512 + (1 - sB) * 256
    k2A = k1A + qA * 128
    s2A = k1A + (1 - qA) * 128
    k2B = k1B + qB * 128
    s2B = k1B + (1 - qB) * 128

    barrier = pltpu.get_barrier_semaphore()
    for nbr in (p1, p2):
        pl.semaphore_signal(
            barrier, inc=1, device_id=(nbr,), device_id_type=pl.DeviceIdType.MESH
        )
    pl.semaphore_wait(barrier, 2)

    def compute_quarter(ro):
        b_off = (ro // SQ) * SQ
        so = ro % SQ
        xq = x_ref[pl.ds(ro, QROWS), :]
        q = jnp.dot(xq, wq_ref[:, :], preferred_element_type=jnp.float32)
        rows = so + lax.broadcasted_iota(jnp.int32, (QROWS, SQ), 0)
        cols = lax.broadcasted_iota(jnp.int32, (QROWS, SQ), 1)
        mask = (cols // BLK) <= (rows // BLK)
        for h in range(HQ_PER):
            qh = q[:, h * DH:(h + 1) * DH]
            kh = k_ref[pl.ds(b_off, SQ), h * DH:(h + 1) * DH]
            s = lax.dot_general(
                qh, kh, (((1,), (1,)), ((), ())),
                preferred_element_type=jnp.float32,
            ) * 0.125
            s = jnp.where(mask, s, -1e9)
            m = jnp.max(s, axis=1, keepdims=True)
            e = jnp.exp(s - m)
            w = e / jnp.sum(e, axis=1, keepdims=True)
            vh = v_ref[pl.ds(b_off, SQ), h * DH:(h + 1) * DH]
            ctx_scr[:, h * DH:(h + 1) * DH] = jnp.dot(
                w, vh, preferred_element_type=jnp.float32
            )
        out_ref[pl.ds(ro, QROWS), :] = jnp.dot(
            ctx_scr[:, :], wo_ref[:, :], preferred_element_type=jnp.float32
        )

    def exch(src_off, n_rows, dst_off, dst_scr, sem, dev):
        dst = out_ref.at[pl.ds(dst_off, n_rows), :] if dst_scr is None else dst_scr
        return pltpu.make_async_remote_copy(
            src_ref=out_ref.at[pl.ds(src_off, n_rows), :],
            dst_ref=dst,
            send_sem=s_sems.at[sem],
            recv_sem=r_sems.at[sem],
            device_id=(dev,),
            device_id_type=pl.DeviceIdType.MESH,
        )

    compute_quarter(s1A)
    rs1A = exch(s1A, 256, None, rs1_scr.at[0], 0, p1)
    rs1A.start()
    compute_quarter(s1B)
    rs1B = exch(s1B, 256, None, rs1_scr.at[1], 1, p2)
    rs1B.start()

    compute_quarter(k1A)
    compute_quarter(k1B)

    rs1A.wait()
    out_ref[pl.ds(k1A, 256), :] += rs1_scr[0, :, :]
    rs1B.wait()
    out_ref[pl.ds(k1B, 256), :] += rs1_scr[1, :, :]

    rs2A = exch(s2A, 128, None, rs2_scr.at[0], 2, p2)
    rs2B = exch(s2B, 128, None, rs2_scr.at[1], 3, p1)
    rs2A.start()
    rs2B.start()
    rs2A.wait()
    out_ref[pl.ds(k2A, 128), :] += rs2_scr[0, :, :]
    rs2B.wait()
    out_ref[pl.ds(k2B, 128), :] += rs2_scr[1, :, :]

    ag1A = exch(k2A, 128, k2A, None, 4, p2)
    ag1B = exch(k2B, 128, k2B, None, 5, p1)
    ag1A.start()
    ag1B.start()
    ag1A.wait()
    ag1B.wait()

    ag2A = exch(k1A, 256, k1A, None, 6, p1)
    ag2B = exch(k1B, 256, k1B, None, 7, p2)
    ag2A.start()
    ag2B.start()
    ag2A.wait()
    ag2B.wait()


def kernel(x, Wq, K_ext, V_ext, Wo):
    B, Sq, D = x.shape
    my = lax.axis_index("i")

    K = lax.dynamic_slice_in_dim(K_ext, my * HQ_PER, HQ_PER, axis=2)
    V = lax.dynamic_slice_in_dim(V_ext, my * HQ_PER, HQ_PER, axis=2)
    K2 = K.reshape(B * Sq, HQ_PER * DH)
    V2 = V.reshape(B * Sq, HQ_PER * DH)
    x2 = x.reshape(B * Sq, D)

    out = pl.pallas_call(
        _body,
        out_shape=jax.ShapeDtypeStruct((B * Sq, D), jnp.float32),
        in_specs=[pl.BlockSpec(memory_space=pltpu.VMEM)] * 5,
        out_specs=pl.BlockSpec(memory_space=pltpu.VMEM),
        scratch_shapes=[
            pltpu.VMEM((QROWS, HQ_PER * DH), jnp.float32),
            pltpu.VMEM((2, 256, D), jnp.float32),
            pltpu.VMEM((2, 128, D), jnp.float32),
            pltpu.SemaphoreType.DMA((8,)),
            pltpu.SemaphoreType.DMA((8,)),
        ],
        compiler_params=pltpu.CompilerParams(collective_id=0),
    )(x2, Wq, K2, V2, Wo)
    return out.reshape(B, Sq, D)


# device time: 45564 ns/iter; 2.8813x vs baseline; 1.2282x over previous
import jax
import jax.numpy as jnp
from jax import lax
from jax.experimental import pallas as pl
from jax.experimental.pallas import tpu as pltpu

N_DEV = 4
HQ_PER = 8
DH = 64
BLK = 64
SQ = 512
QROWS = 256
D = 768

BF16 = jnp.bfloat16
F32 = jnp.float32


def _body(x_ref, wq_ref, k_ref, v_ref, wo_ref, out_ref, ctx_scr,
          snd1, rcv1, snd2, rcv2, sndq, rcvq, rcvf, s_sems, r_sems):
    my = lax.axis_index("i")
    p1 = my ^ 1
    p2 = 3 - my

    sA = ((my + 1) >> 1) & 1
    qA = (my >> 1) & 1
    sB = (my >> 1) & 1
    qB = my & 1

    k1A = sA * 256
    s1A = (1 - sA) * 256
    k1B = 512 + sB * 256
    s1B = 512 + (1 - sB) * 256
    k2A = k1A + qA * 128
    s2A = k1A + (1 - qA) * 128
    k2B = k1B + qB * 128
    s2B = k1B + (1 - qB) * 128

    qA_p1 = (p1 >> 1) & 1
    p1_k2A = s1A + qA_p1 * 128
    p1_s2A = s1A + (1 - qA_p1) * 128
    qB_p2 = p2 & 1
    p2_k2B = s1B + qB_p2 * 128
    p2_s2B = s1B + (1 - qB_p2) * 128

    barrier = pltpu.get_barrier_semaphore()
    for nbr in (p1, p2):
        pl.semaphore_signal(
            barrier, inc=1, device_id=(nbr,), device_id_type=pl.DeviceIdType.MESH
        )
    pl.semaphore_wait(barrier, 2)

    def attn_core(ro, b_off, so, kv):
        xq = x_ref[pl.ds(ro, QROWS), :]
        q = jnp.dot(xq, wq_ref[:, :], preferred_element_type=F32)
        rows = so + lax.broadcasted_iota(jnp.int32, (QROWS, kv), 0)
        cols = lax.broadcasted_iota(jnp.int32, (QROWS, kv), 1)
        mask = (cols // BLK) <= (rows // BLK)
        for h in range(HQ_PER):
            qh = q[:, h * DH:(h + 1) * DH]
            kh = k_ref[pl.ds(b_off, kv), h * DH:(h + 1) * DH]
            s = lax.dot_general(
                qh, kh, (((1,), (1,)), ((), ())),
                preferred_element_type=F32,
            ) * 0.125
            s = jnp.where(mask, s, -1e9)
            m = jnp.max(s, axis=1, keepdims=True)
            e = jnp.exp(s - m)
            w = e / jnp.sum(e, axis=1, keepdims=True)
            vh = v_ref[pl.ds(b_off, kv), h * DH:(h + 1) * DH]
            ctx_scr[:, h * DH:(h + 1) * DH] = jnp.dot(
                w, vh, preferred_element_type=F32
            )
        out_ref[pl.ds(ro, QROWS), :] = jnp.dot(
            ctx_scr[:, :], wo_ref[:, :], preferred_element_type=F32
        )

    def compute_quarter(ro):
        so_dyn = ro % SQ
        b_off = ro - so_dyn

        @pl.when(so_dyn == 0)
        def _():
            attn_core(ro, b_off, 0, 256)

        @pl.when(so_dyn != 0)
        def _():
            attn_core(ro, b_off, 256, SQ)

    def rdma(src, dst, sem, dev):
        return pltpu.make_async_remote_copy(
            src_ref=src, dst_ref=dst,
            send_sem=s_sems.at[sem], recv_sem=r_sems.at[sem],
            device_id=(dev,), device_id_type=pl.DeviceIdType.MESH,
        )

    compute_quarter(s1A)
    snd1[0, :, :] = out_ref[pl.ds(s1A, 256), :].astype(BF16)
    rs1A = rdma(snd1.at[0], rcv1.at[0], 0, p1)
    rs1A.start()
    compute_quarter(s1B)
    snd1[1, :, :] = out_ref[pl.ds(s1B, 256), :].astype(BF16)
    rs1B = rdma(snd1.at[1], rcv1.at[1], 1, p2)
    rs1B.start()
    compute_quarter(k1A)
    compute_quarter(k1B)

    off_sA = (1 - qA) * 128
    off_kA = qA * 128
    rs1A.wait()
    out_ref[pl.ds(s2A, 128), :] += rcv1[0, pl.ds(off_sA, 128), :].astype(F32)
    snd2[0, :, :] = out_ref[pl.ds(s2A, 128), :].astype(BF16)
    rs2A = rdma(snd2.at[0], rcv2.at[0], 2, p2)
    rs2A.start()
    out_ref[pl.ds(k2A, 128), :] += rcv1[0, pl.ds(off_kA, 128), :].astype(F32)

    off_sB = (1 - qB) * 128
    off_kB = qB * 128
    rs1B.wait()
    out_ref[pl.ds(s2B, 128), :] += rcv1[1, pl.ds(off_sB, 128), :].astype(F32)
    snd2[1, :, :] = out_ref[pl.ds(s2B, 128), :].astype(BF16)
    rs2B = rdma(snd2.at[1], rcv2.at[1], 3, p1)
    rs2B.start()
    out_ref[pl.ds(k2B, 128), :] += rcv1[1, pl.ds(off_kB, 128), :].astype(F32)

    rs2A.wait()
    out_ref[pl.ds(k2A, 128), :] += rcv2[0, :, :].astype(F32)
    sndq[0, :, :] = out_ref[pl.ds(k2A, 128), :].astype(BF16)
    ag1A = rdma(sndq.at[0], rcvq.at[0], 4, p2)
    ag1A.start()
    ag2A0 = rdma(sndq.at[0], rcvf.at[0, 0], 6, p1)
    ag2A0.start()

    rs2B.wait()
    out_ref[pl.ds(k2B, 128), :] += rcv2[1, :, :].astype(F32)
    sndq[1, :, :] = out_ref[pl.ds(k2B, 128), :].astype(BF16)
    ag1B = rdma(sndq.at[1], rcvq.at[1], 5, p1)
    ag1B.start()
    ag2B0 = rdma(sndq.at[1], rcvf.at[1, 0], 7, p2)
    ag2B0.start()

    ag1A.wait()
    out_ref[pl.ds(s2A, 128), :] = rcvq[0, :, :].astype(F32)
    ag2A1 = rdma(rcvq.at[0], rcvf.at[0, 1], 8, p1)
    ag2A1.start()
    ag1B.wait()
    out_ref[pl.ds(s2B, 128), :] = rcvq[1, :, :].astype(F32)
    ag2B1 = rdma(rcvq.at[1], rcvf.at[1, 1], 9, p2)
    ag2B1.start()

    ag2A0.wait()
    out_ref[pl.ds(p1_k2A, 128), :] = rcvf[0, 0, :, :].astype(F32)
    ag2A1.wait()
    out_ref[pl.ds(p1_s2A, 128), :] = rcvf[0, 1, :, :].astype(F32)
    ag2B0.wait()
    out_ref[pl.ds(p2_k2B, 128), :] = rcvf[1, 0, :, :].astype(F32)
    ag2B1.wait()
    out_ref[pl.ds(p2_s2B, 128), :] = rcvf[1, 1, :, :].astype(F32)


def kernel(x, Wq, K_ext, V_ext, Wo):
    B, Sq, d = x.shape
    my = lax.axis_index("i")

    K = lax.dynamic_slice_in_dim(K_ext, my * HQ_PER, HQ_PER, axis=2)
    V = lax.dynamic_slice_in_dim(V_ext, my * HQ_PER, HQ_PER, axis=2)
    K2 = K.reshape(B * Sq, HQ_PER * DH)
    V2 = V.reshape(B * Sq, HQ_PER * DH)
    x2 = x.reshape(B * Sq, d)

    out = pl.pallas_call(
        _body,
        out_shape=jax.ShapeDtypeStruct((B * Sq, d), jnp.float32),
        in_specs=[pl.BlockSpec(memory_space=pltpu.VMEM)] * 5,
        out_specs=pl.BlockSpec(memory_space=pltpu.VMEM),
        scratch_shapes=[
            pltpu.VMEM((QROWS, HQ_PER * DH), F32),
            pltpu.VMEM((2, 256, D), BF16),
            pltpu.VMEM((2, 256, D), BF16),
            pltpu.VMEM((2, 128, D), BF16),
            pltpu.VMEM((2, 128, D), BF16),
            pltpu.VMEM((2, 128, D), BF16),
            pltpu.VMEM((2, 128, D), BF16),
            pltpu.VMEM((2, 2, 128, D), BF16),
            pltpu.SemaphoreType.DMA((10,)),
            pltpu.SemaphoreType.DMA((10,)),
        ],
        compiler_params=pltpu.CompilerParams(collective_id=0),
    )(x2, Wq, K2, V2, Wo)
    return out.reshape(B, Sq, d)


# device time: 39144 ns/iter; 3.3539x vs baseline; 1.1640x over previous
import jax
import jax.numpy as jnp
from jax import lax
from jax.experimental import pallas as pl
from jax.experimental.pallas import tpu as pltpu

N_DEV = 4
HQ_PER = 8
DH = 64
BLK = 64
SQ = 512
QROWS = 256
D = 768

BF16 = jnp.bfloat16
F32 = jnp.float32


def _body(x_ref, wq_ref, k_ref, v_ref, wo_ref, out_ref, ctx_scr,
          snd1, rcv1, snd2, rcv2, sndq, rcvq, rcvf, s_sems, r_sems):
    my = lax.axis_index("i")
    p1 = my ^ 1
    p2 = 3 - my

    sA = ((my + 1) >> 1) & 1
    qA = (my >> 1) & 1
    sB = (my >> 1) & 1
    qB = my & 1

    k1A = sA * 256
    s1A = (1 - sA) * 256
    k1B = 512 + sB * 256
    s1B = 512 + (1 - sB) * 256
    k2A = k1A + qA * 128
    s2A = k1A + (1 - qA) * 128
    k2B = k1B + qB * 128
    s2B = k1B + (1 - qB) * 128

    qA_p1 = (p1 >> 1) & 1
    p1_k2A = s1A + qA_p1 * 128
    p1_s2A = s1A + (1 - qA_p1) * 128
    qB_p2 = p2 & 1
    p2_k2B = s1B + qB_p2 * 128
    p2_s2B = s1B + (1 - qB_p2) * 128

    barrier = pltpu.get_barrier_semaphore()
    for nbr in (p1, p2):
        pl.semaphore_signal(
            barrier, inc=1, device_id=(nbr,), device_id_type=pl.DeviceIdType.MESH
        )
    pl.semaphore_wait(barrier, 2)

    def attn_core(ro, b_off, so, kv):
        xq = x_ref[pl.ds(ro, QROWS), :]
        q = jnp.dot(xq, wq_ref[:, :], preferred_element_type=F32)
        q = q * 0.125
        rows = so + lax.broadcasted_iota(jnp.int32, (QROWS, kv), 0)
        cols = lax.broadcasted_iota(jnp.int32, (QROWS, kv), 1)
        mask = (cols // BLK) <= (rows // BLK)
        for h in range(HQ_PER):
            qh = q[:, h * DH:(h + 1) * DH]
            kh = k_ref[pl.ds(b_off, kv), h * DH:(h + 1) * DH]
            s = lax.dot_general(
                qh, kh, (((1,), (1,)), ((), ())),
                preferred_element_type=F32,
            )
            e = jnp.exp(jnp.where(mask, s, -1e9))
            r = 1.0 / jnp.sum(e, axis=1, keepdims=True)
            vh = v_ref[pl.ds(b_off, kv), h * DH:(h + 1) * DH]
            c = jnp.dot(e, vh, preferred_element_type=F32)
            ctx_scr[:, h * DH:(h + 1) * DH] = c * r
        out_ref[pl.ds(ro, QROWS), :] = jnp.dot(
            ctx_scr[:, :], wo_ref[:, :], preferred_element_type=F32
        )

    def compute_quarter(ro):
        so_dyn = ro % SQ
        b_off = ro - so_dyn

        @pl.when(so_dyn == 0)
        def _():
            attn_core(ro, b_off, 0, 256)

        @pl.when(so_dyn != 0)
        def _():
            attn_core(ro, b_off, 256, SQ)

    def rdma(src, dst, sem, dev):
        return pltpu.make_async_remote_copy(
            src_ref=src, dst_ref=dst,
            send_sem=s_sems.at[sem], recv_sem=r_sems.at[sem],
            device_id=(dev,), device_id_type=pl.DeviceIdType.MESH,
        )

    compute_quarter(s1A)
    snd1[0, :, :] = out_ref[pl.ds(s1A, 256), :].astype(BF16)
    rs1A = rdma(snd1.at[0], rcv1.at[0], 0, p1)
    rs1A.start()
    compute_quarter(s1B)
    snd1[1, :, :] = out_ref[pl.ds(s1B, 256), :].astype(BF16)
    rs1B = rdma(snd1.at[1], rcv1.at[1], 1, p2)
    rs1B.start()
    compute_quarter(k1A)

    off_sA = (1 - qA) * 128
    off_kA = qA * 128
    rs1A.wait()
    out_ref[pl.ds(s2A, 128), :] += rcv1[0, pl.ds(off_sA, 128), :].astype(F32)
    snd2[0, :, :] = out_ref[pl.ds(s2A, 128), :].astype(BF16)
    rs2A = rdma(snd2.at[0], rcv2.at[0], 2, p2)
    rs2A.start()
    out_ref[pl.ds(k2A, 128), :] += rcv1[0, pl.ds(off_kA, 128), :].astype(F32)

    compute_quarter(k1B)

    off_sB = (1 - qB) * 128
    off_kB = qB * 128
    rs1B.wait()
    out_ref[pl.ds(s2B, 128), :] += rcv1[1, pl.ds(off_sB, 128), :].astype(F32)
    snd2[1, :, :] = out_ref[pl.ds(s2B, 128), :].astype(BF16)
    rs2B = rdma(snd2.at[1], rcv2.at[1], 3, p1)
    rs2B.start()
    out_ref[pl.ds(k2B, 128), :] += rcv1[1, pl.ds(off_kB, 128), :].astype(F32)

    rs2A.wait()
    out_ref[pl.ds(k2A, 128), :] += rcv2[0, :, :].astype(F32)
    sndq[0, :, :] = out_ref[pl.ds(k2A, 128), :].astype(BF16)
    ag1A = rdma(sndq.at[0], rcvq.at[0], 4, p2)
    ag1A.start()
    ag2A0 = rdma(sndq.at[0], rcvf.at[0, 0], 6, p1)
    ag2A0.start()

    rs2B.wait()
    out_ref[pl.ds(k2B, 128), :] += rcv2[1, :, :].astype(F32)
    sndq[1, :, :] = out_ref[pl.ds(k2B, 128), :].astype(BF16)
    ag1B = rdma(sndq.at[1], rcvq.at[1], 5, p1)
    ag1B.start()
    ag2B0 = rdma(sndq.at[1], rcvf.at[1, 0], 7, p2)
    ag2B0.start()

    ag1A.wait()
    out_ref[pl.ds(s2A, 128), :] = rcvq[0, :, :].astype(F32)
    ag2A1 = rdma(rcvq.at[0], rcvf.at[0, 1], 8, p1)
    ag2A1.start()
    ag1B.wait()
    out_ref[pl.ds(s2B, 128), :] = rcvq[1, :, :].astype(F32)
    ag2B1 = rdma(rcvq.at[1], rcvf.at[1, 1], 9, p2)
    ag2B1.start()

    ag2A0.wait()
    out_ref[pl.ds(p1_k2A, 128), :] = rcvf[0, 0, :, :].astype(F32)
    ag2A1.wait()
    out_ref[pl.ds(p1_s2A, 128), :] = rcvf[0, 1, :, :].astype(F32)
    ag2B0.wait()
    out_ref[pl.ds(p2_k2B, 128), :] = rcvf[1, 0, :, :].astype(F32)
    ag2B1.wait()
    out_ref[pl.ds(p2_s2B, 128), :] = rcvf[1, 1, :, :].astype(F32)


def kernel(x, Wq, K_ext, V_ext, Wo):
    B, Sq, d = x.shape
    my = lax.axis_index("i")

    K = lax.dynamic_slice_in_dim(K_ext, my * HQ_PER, HQ_PER, axis=2)
    V = lax.dynamic_slice_in_dim(V_ext, my * HQ_PER, HQ_PER, axis=2)
    K2 = K.reshape(B * Sq, HQ_PER * DH)
    V2 = V.reshape(B * Sq, HQ_PER * DH)
    x2 = x.reshape(B * Sq, d)

    out = pl.pallas_call(
        _body,
        out_shape=jax.ShapeDtypeStruct((B * Sq, d), jnp.float32),
        in_specs=[pl.BlockSpec(memory_space=pltpu.VMEM)] * 5,
        out_specs=pl.BlockSpec(memory_space=pltpu.VMEM),
        scratch_shapes=[
            pltpu.VMEM((QROWS, HQ_PER * DH), F32),
            pltpu.VMEM((2, 256, D), BF16),
            pltpu.VMEM((2, 256, D), BF16),
            pltpu.VMEM((2, 128, D), BF16),
            pltpu.VMEM((2, 128, D), BF16),
            pltpu.VMEM((2, 128, D), BF16),
            pltpu.VMEM((2, 128, D), BF16),
            pltpu.VMEM((2, 2, 128, D), BF16),
            pltpu.SemaphoreType.DMA((10,)),
            pltpu.SemaphoreType.DMA((10,)),
        ],
        compiler_params=pltpu.CompilerParams(collective_id=0),
    )(x2, Wq, K2, V2, Wo)
    return out.reshape(B, Sq, d)
